# Initial kernel scaffold; baseline (speedup 1.0000x reference)
import jax
import jax.numpy as jnp
from jax.experimental import pallas as pl


def kernel(x, Wg, bg, W1, b1, W2, b2):
    raise NotImplementedError("write your pallas kernel here")


# speedup vs baseline: 4.5752x; 4.5752x over previous
import functools

import jax
import jax.numpy as jnp
from jax.experimental import pallas as pl
from jax.experimental.pallas import tpu as pltpu

_E = 64
_D = 768
_DFF = 128
_T = 4096
_TB = 512
_NB = _T // _TB
_B = 64
_MB = _T // _B + _E
_PMAX = _MB * _B


def _router_kernel(x_ref, wg_ref, bg_ref, sel_ref, rank_ref, gate_ref,
                   counts_ref, cnt_scr):
    i = pl.program_id(0)

    @pl.when(i == 0)
    def _init():
        cnt_scr[...] = jnp.zeros_like(cnt_scr)

    xb = x_ref[...]
    logits = jnp.dot(xb, wg_ref[...],
                     preferred_element_type=jnp.float32) + bg_ref[0]
    m = jnp.max(logits, axis=1, keepdims=True)
    s = jnp.sum(jnp.exp(logits - m), axis=1)
    gate = 1.0 / s
    iota_e = jax.lax.broadcasted_iota(jnp.int32, (_TB, _E), 1)
    eq = logits == m
    sel = jnp.min(jnp.where(eq, iota_e, _E), axis=1)
    oh = (iota_e == sel[:, None]).astype(jnp.float32)

    r = jax.lax.broadcasted_iota(jnp.int32, (_TB, _TB), 0)
    c = jax.lax.broadcasted_iota(jnp.int32, (_TB, _TB), 1)
    tri = (r > c).astype(jnp.float32)
    pref = jnp.dot(tri, oh, preferred_element_type=jnp.float32)

    base = cnt_scr[0]
    rank = jnp.sum((pref + base[None, :]) * oh, axis=1)
    new_counts = base + jnp.sum(oh, axis=0)
    cnt_scr[0] = new_counts

    sel_ref[0, 0] = sel
    rank_ref[0, 0] = rank.astype(jnp.int32)
    gate_ref[0, 0] = gate
    counts_ref[0] = new_counts.astype(jnp.int32)


def _router(x, Wg, bg):
    return pl.pallas_call(
        _router_kernel,
        grid=(_NB,),
        in_specs=[
            pl.BlockSpec((_TB, _D), lambda i: (i, 0)),
            pl.BlockSpec((_D, _E), lambda i: (0, 0)),
            pl.BlockSpec((1, _E), lambda i: (0, 0)),
        ],
        out_specs=[
            pl.BlockSpec((1, 1, _TB), lambda i: (i, 0, 0)),
            pl.BlockSpec((1, 1, _TB), lambda i: (i, 0, 0)),
            pl.BlockSpec((1, 1, _TB), lambda i: (i, 0, 0)),
            pl.BlockSpec((1, _E), lambda i: (0, 0)),
        ],
        out_shape=[
            jax.ShapeDtypeStruct((_NB, 1, _TB), jnp.int32),
            jax.ShapeDtypeStruct((_NB, 1, _TB), jnp.int32),
            jax.ShapeDtypeStruct((_NB, 1, _TB), jnp.float32),
            jax.ShapeDtypeStruct((1, _E), jnp.int32),
        ],
        scratch_shapes=[pltpu.VMEM((8, _E), jnp.float32)],
        compiler_params=pltpu.CompilerParams(
            dimension_semantics=("arbitrary",)),
    )(x, Wg, bg.reshape(1, _E))


def _ffn_kernel(be_ref, nu_ref, xs_ref, gs_ref, w1_ref, b1_ref, w2_ref,
                b2_ref, ys_ref):
    i = pl.program_id(0)

    @pl.when(i < nu_ref[0])
    def _body():
        xb = xs_ref[...]
        h = jnp.maximum(
            jnp.dot(xb, w1_ref[0], preferred_element_type=jnp.float32)
            + b1_ref[0], 0.0)
        y = (jnp.dot(h, w2_ref[0], preferred_element_type=jnp.float32)
             + b2_ref[0])
        ys_ref[...] = y * gs_ref[0, 0][:, None]


def _grouped_ffn(xs, gs, W1, b1, W2, b2, block_expert, num_used):
    grid_spec = pltpu.PrefetchScalarGridSpec(
        num_scalar_prefetch=2,
        grid=(_MB,),
        in_specs=[
            pl.BlockSpec((_B, _D),
                         lambda i, be, nu: (jnp.minimum(i, nu[0] - 1), 0)),
            pl.BlockSpec((1, 1, _B),
                         lambda i, be, nu: (jnp.minimum(i, nu[0] - 1), 0, 0)),
            pl.BlockSpec((1, _D, _DFF), lambda i, be, nu: (be[i], 0, 0)),
            pl.BlockSpec((1, 1, _DFF), lambda i, be, nu: (be[i], 0, 0)),
            pl.BlockSpec((1, _DFF, _D), lambda i, be, nu: (be[i], 0, 0)),
            pl.BlockSpec((1, 1, _D), lambda i, be, nu: (be[i], 0, 0)),
        ],
        out_specs=pl.BlockSpec(
            (_B, _D), lambda i, be, nu: (jnp.minimum(i, nu[0] - 1), 0)),
    )
    return pl.pallas_call(
        _ffn_kernel,
        grid_spec=grid_spec,
        out_shape=jax.ShapeDtypeStruct((_PMAX, _D), jnp.float32),
        compiler_params=pltpu.CompilerParams(
            dimension_semantics=("arbitrary",)),
    )(block_expert, num_used, xs, gs.reshape(_MB, 1, _B),
      W1, b1.reshape(_E, 1, _DFF), W2, b2.reshape(_E, 1, _D))


def kernel(x, Wg, bg, W1, b1, W2, b2):
    sel3, rank3, gate3, counts2 = _router(x, Wg, bg)
    sel = sel3.reshape(_T)
    rank = rank3.reshape(_T)
    gate = gate3.reshape(_T)
    counts = counts2.reshape(_E)

    blocks_per_e = (counts + (_B - 1)) // _B
    num_used = jnp.sum(blocks_per_e).astype(jnp.int32).reshape(1)
    block_expert = jnp.repeat(
        jnp.arange(_E, dtype=jnp.int32), blocks_per_e,
        total_repeat_length=_MB)
    block_expert = jnp.minimum(block_expert, _E - 1)
    offsets = (_B * (jnp.cumsum(blocks_per_e) - blocks_per_e)).astype(jnp.int32)

    dest = offsets[sel] + rank
    xs = jnp.zeros((_PMAX, _D), jnp.float32).at[dest].set(x)
    gs = jnp.zeros((_PMAX,), jnp.float32).at[dest].set(gate)

    ys = _grouped_ffn(xs, gs, W1, b1, W2, b2, block_expert, num_used)
    return ys[dest]


# speedup vs baseline: 6.8111x; 1.4887x over previous
import functools

import jax
import jax.numpy as jnp
from jax import lax
from jax.experimental import pallas as pl
from jax.experimental.pallas import tpu as pltpu
from jax.experimental.pallas import tpu_sc as plsc

_E = 64
_D = 768
_DFF = 128
_T = 4096
_TB = 512
_NB = _T // _TB
_B = 64
_MB = _T // _B + _E
_PMAX = _MB * _B

_NC = 2
_NS = 16
_NW = _NC * _NS
_TPW = _T // _NW


def _router_kernel(x_ref, wg_ref, bg_ref, sel_ref, rank_ref, gate_ref,
                   counts_ref, cnt_scr):
    i = pl.program_id(0)

    @pl.when(i == 0)
    def _init():
        cnt_scr[...] = jnp.zeros_like(cnt_scr)

    xb = x_ref[...]
    logits = jnp.dot(xb, wg_ref[...],
                     preferred_element_type=jnp.float32) + bg_ref[0]
    m = jnp.max(logits, axis=1, keepdims=True)
    s = jnp.sum(jnp.exp(logits - m), axis=1)
    gate = 1.0 / s
    iota_e = jax.lax.broadcasted_iota(jnp.int32, (_TB, _E), 1)
    eq = logits == m
    sel = jnp.min(jnp.where(eq, iota_e, _E), axis=1)
    oh = (iota_e == sel[:, None]).astype(jnp.float32)

    r = jax.lax.broadcasted_iota(jnp.int32, (_TB, _TB), 0)
    c = jax.lax.broadcasted_iota(jnp.int32, (_TB, _TB), 1)
    tri = (r > c).astype(jnp.float32)
    pref = jnp.dot(tri, oh, preferred_element_type=jnp.float32)

    base = cnt_scr[0]
    rank = jnp.sum((pref + base[None, :]) * oh, axis=1)
    new_counts = base + jnp.sum(oh, axis=0)
    cnt_scr[0] = new_counts

    sel_ref[0, 0] = sel
    rank_ref[0, 0] = rank.astype(jnp.int32)
    gate_ref[0, 0] = gate
    counts_ref[0] = new_counts.astype(jnp.int32)


def _router(x, Wg, bg):
    return pl.pallas_call(
        _router_kernel,
        grid=(_NB,),
        in_specs=[
            pl.BlockSpec((_TB, _D), lambda i: (i, 0)),
            pl.BlockSpec((_D, _E), lambda i: (0, 0)),
            pl.BlockSpec((1, _E), lambda i: (0, 0)),
        ],
        out_specs=[
            pl.BlockSpec((1, 1, _TB), lambda i: (i, 0, 0)),
            pl.BlockSpec((1, 1, _TB), lambda i: (i, 0, 0)),
            pl.BlockSpec((1, 1, _TB), lambda i: (i, 0, 0)),
            pl.BlockSpec((1, _E), lambda i: (0, 0)),
        ],
        out_shape=[
            jax.ShapeDtypeStruct((_NB, 1, _TB), jnp.int32),
            jax.ShapeDtypeStruct((_NB, 1, _TB), jnp.int32),
            jax.ShapeDtypeStruct((_NB, 1, _TB), jnp.float32),
            jax.ShapeDtypeStruct((1, _E), jnp.int32),
        ],
        scratch_shapes=[pltpu.VMEM((8, _E), jnp.float32)],
        compiler_params=pltpu.CompilerParams(
            dimension_semantics=("arbitrary",)),
    )(x, Wg, bg.reshape(1, _E))


def _ffn_kernel(be_ref, nu_ref, xs_ref, gs_ref, w1_ref, b1_ref, w2_ref,
                b2_ref, ys_ref):
    i = pl.program_id(0)

    @pl.when(i < nu_ref[0])
    def _body():
        xb = xs_ref[...]
        h = jnp.maximum(
            jnp.dot(xb, w1_ref[0], preferred_element_type=jnp.float32)
            + b1_ref[0], 0.0)
        y = (jnp.dot(h, w2_ref[0], preferred_element_type=jnp.float32)
             + b2_ref[0])
        ys_ref[...] = y * gs_ref[0, :, 0][:, None]


def _grouped_ffn(xs, gs, W1, b1, W2, b2, block_expert, num_used):
    grid_spec = pltpu.PrefetchScalarGridSpec(
        num_scalar_prefetch=2,
        grid=(_MB,),
        in_specs=[
            pl.BlockSpec((_B, _D),
                         lambda i, be, nu: (jnp.minimum(i, nu[0] - 1), 0)),
            pl.BlockSpec((1, _B, 128),
                         lambda i, be, nu: (jnp.minimum(i, nu[0] - 1), 0, 0)),
            pl.BlockSpec((1, _D, _DFF), lambda i, be, nu: (be[i], 0, 0)),
            pl.BlockSpec((1, 1, _DFF), lambda i, be, nu: (be[i], 0, 0)),
            pl.BlockSpec((1, _DFF, _D), lambda i, be, nu: (be[i], 0, 0)),
            pl.BlockSpec((1, 1, _D), lambda i, be, nu: (be[i], 0, 0)),
        ],
        out_specs=pl.BlockSpec(
            (_B, _D), lambda i, be, nu: (jnp.minimum(i, nu[0] - 1), 0)),
    )
    return pl.pallas_call(
        _ffn_kernel,
        grid_spec=grid_spec,
        out_shape=jax.ShapeDtypeStruct((_PMAX, _D), jnp.float32),
        compiler_params=pltpu.CompilerParams(
            dimension_semantics=("arbitrary",)),
    )(block_expert, num_used, xs, gs.reshape(_MB, _B, 128),
      W1, b1.reshape(_E, 1, _DFF), W2, b2.reshape(_E, 1, _D))


def _sc_mesh():
    return plsc.VectorSubcoreMesh(core_axis_name="c", subcore_axis_name="s")


def _compute_dest(sel_v, rank_v, off_v, dest_v):
    for j in range(_TPW // 16):
        sv = sel_v[pl.ds(16 * j, 16)]
        rv = rank_v[pl.ds(16 * j, 16)]
        off = plsc.load_gather(off_v, [sv])
        dest_v[pl.ds(16 * j, 16)] = off + rv


@functools.partial(
    pl.kernel,
    mesh=_sc_mesh(),
    out_type=[
        jax.ShapeDtypeStruct((_PMAX, _D), jnp.float32),
        jax.ShapeDtypeStruct((_PMAX, 128), jnp.float32),
    ],
    scratch_types=[
        pltpu.VMEM((_TPW,), jnp.int32),
        pltpu.VMEM((_TPW,), jnp.int32),
        pltpu.VMEM((128,), jnp.int32),
        pltpu.VMEM((_TPW,), jnp.int32),
        pltpu.VMEM((_TPW, _D), jnp.float32),
        pltpu.VMEM((_TPW,), jnp.float32),
        pltpu.VMEM((_TPW, 128), jnp.float32),
        pltpu.SemaphoreType.DMA,
        pltpu.SemaphoreType.DMA,
    ],
    compiler_params=pltpu.CompilerParams(needs_layout_passes=False),
)
def _sc_dispatch(x_hbm, sel_hbm, rank_hbm, off_hbm, gate_hbm, xs_hbm, gs_hbm,
                 sel_v, rank_v, off_v, dest_v, rows_v, gate_v, grows_v,
                 sem_x, sem_g):
    wid = lax.axis_index("s") * _NC + lax.axis_index("c")
    base = wid * _TPW
    pltpu.sync_copy(sel_hbm.at[pl.ds(base, _TPW)], sel_v)
    pltpu.sync_copy(rank_hbm.at[pl.ds(base, _TPW)], rank_v)
    pltpu.sync_copy(off_hbm, off_v.at[pl.ds(0, _E)])
    pltpu.sync_copy(gate_hbm.at[pl.ds(base, _TPW)], gate_v)
    pltpu.sync_copy(x_hbm.at[pl.ds(base, _TPW)], rows_v)
    _compute_dest(sel_v, rank_v, off_v, dest_v)

    def _gate_row(t, carry):
        idx = jnp.full((16,), t, jnp.int32)
        g = plsc.load_gather(gate_v, [idx])
        for j in range(8):
            grows_v[t, pl.ds(16 * j, 16)] = g
        return carry

    lax.fori_loop(0, _TPW, _gate_row, 0)

    cp_x = pltpu.async_copy(rows_v, xs_hbm.at[dest_v], sem_x)
    cp_g = pltpu.async_copy(grows_v, gs_hbm.at[dest_v], sem_g)
    cp_x.wait()
    cp_g.wait()


@functools.partial(
    pl.kernel,
    mesh=_sc_mesh(),
    out_type=jax.ShapeDtypeStruct((_T, _D), jnp.float32),
    scratch_types=[
        pltpu.VMEM((_TPW,), jnp.int32),
        pltpu.VMEM((_TPW,), jnp.int32),
        pltpu.VMEM((128,), jnp.int32),
        pltpu.VMEM((_TPW,), jnp.int32),
        pltpu.VMEM((_TPW, _D), jnp.float32),
        pltpu.SemaphoreType.DMA,
    ],
    compiler_params=pltpu.CompilerParams(needs_layout_passes=False),
)
def _sc_combine(ys_hbm, sel_hbm, rank_hbm, off_hbm, out_hbm,
                sel_v, rank_v, off_v, dest_v, rows_v, sem):
    wid = lax.axis_index("s") * _NC + lax.axis_index("c")
    base = wid * _TPW
    pltpu.sync_copy(sel_hbm.at[pl.ds(base, _TPW)], sel_v)
    pltpu.sync_copy(rank_hbm.at[pl.ds(base, _TPW)], rank_v)
    pltpu.sync_copy(off_hbm, off_v.at[pl.ds(0, _E)])
    _compute_dest(sel_v, rank_v, off_v, dest_v)
    pltpu.async_copy(ys_hbm.at[dest_v], rows_v, sem).wait()
    pltpu.sync_copy(rows_v, out_hbm.at[pl.ds(base, _TPW)])


def kernel(x, Wg, bg, W1, b1, W2, b2):
    sel3, rank3, gate3, counts2 = _router(x, Wg, bg)
    sel = sel3.reshape(_T)
    rank = rank3.reshape(_T)
    gate = gate3.reshape(_T)
    counts = counts2.reshape(_E)

    blocks_per_e = (counts + (_B - 1)) // _B
    num_used = jnp.sum(blocks_per_e).astype(jnp.int32).reshape(1)
    block_expert = jnp.repeat(
        jnp.arange(_E, dtype=jnp.int32), blocks_per_e,
        total_repeat_length=_MB)
    block_expert = jnp.minimum(block_expert, _E - 1)
    offsets = (_B * (jnp.cumsum(blocks_per_e) - blocks_per_e)).astype(jnp.int32)

    xs, gs16 = _sc_dispatch(x, sel, rank, offsets, gate)

    ys = _grouped_ffn(xs, gs16, W1, b1, W2, b2, block_expert, num_used)

    return _sc_combine(ys, sel, rank, offsets)


# speedup vs baseline: 7.6566x; 1.1241x over previous
import functools

import jax
import jax.numpy as jnp
from jax import lax
from jax.experimental import pallas as pl
from jax.experimental.pallas import tpu as pltpu
from jax.experimental.pallas import tpu_sc as plsc

_E = 64
_D = 768
_DFF = 128
_T = 4096
_TB = 512
_NB = _T // _TB
_B = 128
_MB = _T // _B + _E
_PMAX = _MB * _B

_NC = 2
_NS = 16
_NW = _NC * _NS
_TPW = _T // _NW


def _router_kernel(x_ref, wg_ref, bg_ref, sel_ref, rank_ref, gate_ref,
                   counts_ref, cnt_scr):
    i = pl.program_id(0)

    @pl.when(i == 0)
    def _init():
        cnt_scr[...] = jnp.zeros_like(cnt_scr)

    xb = x_ref[...]
    logits = jnp.dot(xb, wg_ref[...],
                     preferred_element_type=jnp.float32) + bg_ref[0]
    m = jnp.max(logits, axis=1, keepdims=True)
    s = jnp.sum(jnp.exp(logits - m), axis=1)
    gate = 1.0 / s
    iota_e = jax.lax.broadcasted_iota(jnp.int32, (_TB, _E), 1)
    eq = logits == m
    sel = jnp.min(jnp.where(eq, iota_e, _E), axis=1)
    oh = (iota_e == sel[:, None]).astype(jnp.float32)

    r = jax.lax.broadcasted_iota(jnp.int32, (_TB, _TB), 0)
    c = jax.lax.broadcasted_iota(jnp.int32, (_TB, _TB), 1)
    tri = (r > c).astype(jnp.float32)
    pref = jnp.dot(tri, oh, preferred_element_type=jnp.float32)

    base = cnt_scr[0]
    rank = jnp.sum((pref + base[None, :]) * oh, axis=1)
    new_counts = base + jnp.sum(oh, axis=0)
    cnt_scr[0] = new_counts

    sel_ref[0, 0] = sel
    rank_ref[0, 0] = rank.astype(jnp.int32)
    gate_ref[0, 0] = gate
    counts_ref[0] = new_counts.astype(jnp.int32)


def _router(x, Wg, bg):
    return pl.pallas_call(
        _router_kernel,
        grid=(_NB,),
        in_specs=[
            pl.BlockSpec((_TB, _D), lambda i: (i, 0)),
            pl.BlockSpec((_D, _E), lambda i: (0, 0)),
            pl.BlockSpec((1, _E), lambda i: (0, 0)),
        ],
        out_specs=[
            pl.BlockSpec((1, 1, _TB), lambda i: (i, 0, 0)),
            pl.BlockSpec((1, 1, _TB), lambda i: (i, 0, 0)),
            pl.BlockSpec((1, 1, _TB), lambda i: (i, 0, 0)),
            pl.BlockSpec((1, _E), lambda i: (0, 0)),
        ],
        out_shape=[
            jax.ShapeDtypeStruct((_NB, 1, _TB), jnp.int32),
            jax.ShapeDtypeStruct((_NB, 1, _TB), jnp.int32),
            jax.ShapeDtypeStruct((_NB, 1, _TB), jnp.float32),
            jax.ShapeDtypeStruct((1, _E), jnp.int32),
        ],
        scratch_shapes=[pltpu.VMEM((8, _E), jnp.float32)],
        compiler_params=pltpu.CompilerParams(
            dimension_semantics=("arbitrary",)),
    )(x, Wg, bg.reshape(1, _E))


def _ffn_kernel(be_ref, nu_ref, xs_ref, gs_ref, w1_ref, b1_ref, w2_ref,
                b2_ref, ys_ref):
    i = pl.program_id(0)

    @pl.when(i < nu_ref[0])
    def _body():
        xb = xs_ref[...]
        h = jnp.maximum(
            jnp.dot(xb, w1_ref[0], preferred_element_type=jnp.float32)
            + b1_ref[0], 0.0)
        y = (jnp.dot(h, w2_ref[0], preferred_element_type=jnp.float32)
             + b2_ref[0])
        ys_ref[...] = y * gs_ref[0, :, 0][:, None]


def _grouped_ffn(xs, gs, W1, b1, W2, b2, block_expert, num_used):
    grid_spec = pltpu.PrefetchScalarGridSpec(
        num_scalar_prefetch=2,
        grid=(_MB,),
        in_specs=[
            pl.BlockSpec((_B, _D),
                         lambda i, be, nu: (jnp.minimum(i, nu[0] - 1), 0)),
            pl.BlockSpec((1, _B, 128),
                         lambda i, be, nu: (jnp.minimum(i, nu[0] - 1), 0, 0)),
            pl.BlockSpec((1, _D, _DFF), lambda i, be, nu: (be[i], 0, 0)),
            pl.BlockSpec((1, 1, _DFF), lambda i, be, nu: (be[i], 0, 0)),
            pl.BlockSpec((1, _DFF, _D), lambda i, be, nu: (be[i], 0, 0)),
            pl.BlockSpec((1, 1, _D), lambda i, be, nu: (be[i], 0, 0)),
        ],
        out_specs=pl.BlockSpec(
            (_B, _D), lambda i, be, nu: (jnp.minimum(i, nu[0] - 1), 0)),
    )
    return pl.pallas_call(
        _ffn_kernel,
        grid_spec=grid_spec,
        out_shape=jax.ShapeDtypeStruct((_PMAX, _D), jnp.float32),
        compiler_params=pltpu.CompilerParams(
            dimension_semantics=("arbitrary",)),
    )(block_expert, num_used, xs, gs.reshape(_MB, _B, 128),
      W1, b1.reshape(_E, 1, _DFF), W2, b2.reshape(_E, 1, _D))


def _sc_mesh():
    return plsc.VectorSubcoreMesh(core_axis_name="c", subcore_axis_name="s")


def _compute_dest(sel_v, rank_v, off_v, dest_v):
    for j in range(_TPW // 16):
        sv = sel_v[pl.ds(16 * j, 16)]
        rv = rank_v[pl.ds(16 * j, 16)]
        off = plsc.load_gather(off_v, [sv])
        dest_v[pl.ds(16 * j, 16)] = off + rv


@functools.partial(
    pl.kernel,
    mesh=_sc_mesh(),
    out_type=[
        jax.ShapeDtypeStruct((_PMAX, _D), jnp.float32),
        jax.ShapeDtypeStruct((_PMAX, 128), jnp.float32),
    ],
    scratch_types=[
        pltpu.VMEM((_TPW,), jnp.int32),
        pltpu.VMEM((_TPW,), jnp.int32),
        pltpu.VMEM((128,), jnp.int32),
        pltpu.VMEM((_TPW,), jnp.int32),
        pltpu.VMEM((_TPW, _D), jnp.float32),
        pltpu.VMEM((_TPW,), jnp.float32),
        pltpu.VMEM((_TPW, 128), jnp.float32),
        pltpu.SemaphoreType.DMA,
        pltpu.SemaphoreType.DMA,
    ],
    compiler_params=pltpu.CompilerParams(needs_layout_passes=False),
)
def _sc_dispatch(x_hbm, sel_hbm, rank_hbm, off_hbm, gate_hbm, xs_hbm, gs_hbm,
                 sel_v, rank_v, off_v, dest_v, rows_v, gate_v, grows_v,
                 sem_x, sem_g):
    wid = lax.axis_index("s") * _NC + lax.axis_index("c")
    base = wid * _TPW
    pltpu.sync_copy(sel_hbm.at[pl.ds(base, _TPW)], sel_v)
    pltpu.sync_copy(rank_hbm.at[pl.ds(base, _TPW)], rank_v)
    pltpu.sync_copy(off_hbm, off_v.at[pl.ds(0, _E)])
    pltpu.sync_copy(gate_hbm.at[pl.ds(base, _TPW)], gate_v)
    pltpu.sync_copy(x_hbm.at[pl.ds(base, _TPW)], rows_v)
    _compute_dest(sel_v, rank_v, off_v, dest_v)

    def _gate_row(t, carry):
        idx = jnp.full((16,), t, jnp.int32)
        g = plsc.load_gather(gate_v, [idx])
        for j in range(8):
            grows_v[t, pl.ds(16 * j, 16)] = g
        return carry

    lax.fori_loop(0, _TPW, _gate_row, 0)

    cp_x = pltpu.async_copy(rows_v, xs_hbm.at[dest_v], sem_x)
    cp_g = pltpu.async_copy(grows_v, gs_hbm.at[dest_v], sem_g)
    cp_x.wait()
    cp_g.wait()


@functools.partial(
    pl.kernel,
    mesh=_sc_mesh(),
    out_type=jax.ShapeDtypeStruct((_T, _D), jnp.float32),
    scratch_types=[
        pltpu.VMEM((_TPW,), jnp.int32),
        pltpu.VMEM((_TPW,), jnp.int32),
        pltpu.VMEM((128,), jnp.int32),
        pltpu.VMEM((_TPW,), jnp.int32),
        pltpu.VMEM((_TPW, _D), jnp.float32),
        pltpu.SemaphoreType.DMA,
    ],
    compiler_params=pltpu.CompilerParams(needs_layout_passes=False),
)
def _sc_combine(ys_hbm, sel_hbm, rank_hbm, off_hbm, out_hbm,
                sel_v, rank_v, off_v, dest_v, rows_v, sem):
    wid = lax.axis_index("s") * _NC + lax.axis_index("c")
    base = wid * _TPW
    pltpu.sync_copy(sel_hbm.at[pl.ds(base, _TPW)], sel_v)
    pltpu.sync_copy(rank_hbm.at[pl.ds(base, _TPW)], rank_v)
    pltpu.sync_copy(off_hbm, off_v.at[pl.ds(0, _E)])
    _compute_dest(sel_v, rank_v, off_v, dest_v)
    pltpu.async_copy(ys_hbm.at[dest_v], rows_v, sem).wait()
    pltpu.sync_copy(rows_v, out_hbm.at[pl.ds(base, _TPW)])


def kernel(x, Wg, bg, W1, b1, W2, b2):
    sel3, rank3, gate3, counts2 = _router(x, Wg, bg)
    sel = sel3.reshape(_T)
    rank = rank3.reshape(_T)
    gate = gate3.reshape(_T)
    counts = counts2.reshape(_E)

    blocks_per_e = (counts + (_B - 1)) // _B
    num_used = jnp.sum(blocks_per_e).astype(jnp.int32).reshape(1)
    block_expert = jnp.repeat(
        jnp.arange(_E, dtype=jnp.int32), blocks_per_e,
        total_repeat_length=_MB)
    block_expert = jnp.minimum(block_expert, _E - 1)
    offsets = (_B * (jnp.cumsum(blocks_per_e) - blocks_per_e)).astype(jnp.int32)

    xs, gs16 = _sc_dispatch(x, sel, rank, offsets, gate)

    ys = _grouped_ffn(xs, gs16, W1, b1, W2, b2, block_expert, num_used)

    return _sc_combine(ys, sel, rank, offsets)
